# Initial kernel scaffold; baseline (speedup 1.0000x reference)
import jax
import jax.numpy as jnp
from jax.experimental import pallas as pl


def kernel(user_table, item_table, edge_index):
    raise NotImplementedError("write your pallas kernel here")


# speedup vs baseline: 12.5425x; 12.5425x over previous
import functools

import jax
import jax.numpy as jnp
from jax import lax
from jax.experimental import pallas as pl
from jax.experimental.pallas import tpu as pltpu
from jax.experimental.pallas import tpu_sc as plsc

_NUM_USERS = 25000
_NUM_ITEMS = 25000
_D = 64
_HALF = 32
_N = _NUM_USERS + _NUM_ITEMS
_E = 800000
_NS = 16
_NPAD = 51200
_RPT = _NPAD // _NS
_WCH = 128
_NWCH = _RPT // _WCH
_CHUNK = 80
_EPT = _E // _NS
_BLK = 25
_NBLK = _EPT // (_CHUNK * _BLK)
_NCHROWS = _E // _CHUNK


def _lgcn_body(x0, src3, dst2, out_sum, za, zb,
               acc, dacc, ones1, srcb, dstb, rows0, rows1, wbuf, sbuf,
               dinv, sg0, sg1):
    c = lax.axis_index("c")
    s = lax.axis_index("s")
    row0 = s * _RPT
    nbase = c * _NPAD + row0
    blk0_d = s * _NBLK
    blk0_s = c * (_NS * _NBLK) + blk0_d

    f1 = jnp.full((16,), 1.0, jnp.float32)
    f0 = jnp.zeros((16,), jnp.float32)

    def _fill(ref, nrows, vec):
        def f(r, _):
            ref[r, 0:16] = vec
            ref[r, 16:32] = vec
            return 0
        lax.fori_loop(0, nrows, f, 0)

    def _clear_acc_slice():
        _fill(wbuf, _WCH, f0)
        def f(w, _):
            pltpu.sync_copy(wbuf, acc.at[pl.ds(row0 + w * _WCH, _WCH)])
            return 0
        lax.fori_loop(0, _NWCH, f, 0)

    def _edge_pass(zsrc):
        def blk(b, _):
            pltpu.sync_copy(dst2.at[blk0_d + b], dstb)
            if zsrc is None:
                for j in range(_BLK):
                    pltpu.sync_copy(ones1, dacc.at[dstb.at[j]], add=True)
            else:
                pltpu.sync_copy(src3.at[blk0_s + b], srcb)
                bufs = (rows0, rows1)
                sems = (sg0, sg1)
                pending = pltpu.async_copy(zsrc.at[srcb.at[0]], rows0, sg0)
                for j in range(_BLK):
                    pending.wait()
                    if j + 1 < _BLK:
                        pending = pltpu.async_copy(
                            zsrc.at[srcb.at[j + 1]], bufs[(j + 1) % 2],
                            sems[(j + 1) % 2])
                    pltpu.sync_copy(bufs[j % 2], acc.at[dstb.at[j]], add=True)
            return 0
        lax.fori_loop(0, _NBLK, blk, 0)

    def _dinv_phase():
        magic = jnp.full((16,), 0x5F3759DF, jnp.int32)
        one_i = jnp.full((16,), 1, jnp.int32)
        pltpu.sync_copy(dacc.at[pl.ds(row0, _RPT)], dinv)
        def gf(g, _):
            d = dinv[pl.ds(g * 16, 16)]
            ib = lax.bitcast_convert_type(d, jnp.int32)
            y = lax.bitcast_convert_type(
                magic - lax.shift_right_logical(ib, one_i), jnp.float32)
            for _i in range(4):
                y = y * (1.5 - 0.5 * d * y * y)
            y = jnp.where(d > 0.5, y, 0.0)
            dinv[pl.ds(g * 16, 16)] = y
            return 0
        lax.fori_loop(0, _RPT // 16, gf, 0)

    def _z0_phase():
        def wchunk(w, _):
            nb = nbase + w * _WCH
            pltpu.sync_copy(x0.at[pl.ds(nb, _WCH)], wbuf)
            pltpu.sync_copy(wbuf, out_sum.at[pl.ds(nb, _WCH)])
            def gf(g, _):
                dvec = dinv[pl.ds(w * _WCH + g * 16, 16)]
                for r16 in range(16):
                    r = g * 16 + r16
                    di = dvec[r16]
                    wbuf[r, 0:16] = wbuf[r, 0:16] * di
                    wbuf[r, 16:32] = wbuf[r, 16:32] * di
                return 0
            lax.fori_loop(0, _WCH // 16, gf, 0)
            pltpu.sync_copy(wbuf, za.at[pl.ds(nb, _WCH)])
            return 0
        lax.fori_loop(0, _NWCH, wchunk, 0)

    def _writeback(last, zdst):
        def wchunk(w, _):
            nb = nbase + w * _WCH
            pltpu.sync_copy(acc.at[pl.ds(row0 + w * _WCH, _WCH)], wbuf)
            pltpu.sync_copy(out_sum.at[pl.ds(nb, _WCH)], sbuf)
            def gf(g, _):
                dvec = dinv[pl.ds(w * _WCH + g * 16, 16)]
                for r16 in range(16):
                    r = g * 16 + r16
                    di = dvec[r16]
                    s0 = sbuf[r, 0:16] + wbuf[r, 0:16] * di
                    s1 = sbuf[r, 16:32] + wbuf[r, 16:32] * di
                    if last:
                        sbuf[r, 0:16] = s0 * 0.25
                        sbuf[r, 16:32] = s1 * 0.25
                    else:
                        sbuf[r, 0:16] = s0
                        sbuf[r, 16:32] = s1
                        d2 = di * di
                        wbuf[r, 0:16] = wbuf[r, 0:16] * d2
                        wbuf[r, 16:32] = wbuf[r, 16:32] * d2
                return 0
            lax.fori_loop(0, _WCH // 16, gf, 0)
            pltpu.sync_copy(sbuf, out_sum.at[pl.ds(nb, _WCH)])
            if not last:
                pltpu.sync_copy(wbuf, zdst.at[pl.ds(nb, _WCH)])
            return 0
        lax.fori_loop(0, _NWCH, wchunk, 0)

    for k in range(_CHUNK // 16):
        ones1[pl.ds(k * 16, 16)] = f1
    def zf(g, _):
        dinv[pl.ds(g * 16, 16)] = f0
        return 0
    lax.fori_loop(0, _RPT // 16, zf, 0)
    pltpu.sync_copy(dinv, dacc.at[pl.ds(row0, _RPT)])
    plsc.subcore_barrier()
    _edge_pass(None)
    plsc.subcore_barrier()
    _dinv_phase()
    _z0_phase()

    zsrc = za
    for l in range(3):
        _clear_acc_slice()
        plsc.subcore_barrier()
        _edge_pass(zsrc)
        plsc.subcore_barrier()
        zdst = zb if zsrc is za else za
        _writeback(last=(l == 2), zdst=zdst)
        zsrc = zdst


_lgcn = functools.partial(
    pl.kernel,
    out_type=(
        jax.ShapeDtypeStruct((2 * _NPAD, _HALF), jnp.float32),
        jax.ShapeDtypeStruct((2 * _NPAD, _HALF), jnp.float32),
        jax.ShapeDtypeStruct((2 * _NPAD, _HALF), jnp.float32),
    ),
    mesh=plsc.VectorSubcoreMesh(core_axis_name="c", subcore_axis_name="s"),
    compiler_params=pltpu.CompilerParams(use_tc_tiling_on_sc=False),
    scratch_types=[
        pltpu.VMEM_SHARED((_NPAD, _HALF), jnp.float32),
        pltpu.VMEM_SHARED((_NPAD,), jnp.float32),
        pltpu.VMEM((_CHUNK,), jnp.float32),
        pltpu.VMEM((_BLK, _CHUNK), jnp.int32),
        pltpu.VMEM((_BLK, _CHUNK), jnp.int32),
        pltpu.VMEM((_CHUNK, _HALF), jnp.float32),
        pltpu.VMEM((_CHUNK, _HALF), jnp.float32),
        pltpu.VMEM((_WCH, _HALF), jnp.float32),
        pltpu.VMEM((_WCH, _HALF), jnp.float32),
        pltpu.VMEM((_RPT,), jnp.float32),
        pltpu.SemaphoreType.DMA,
        pltpu.SemaphoreType.DMA,
    ],
)(_lgcn_body)


def kernel(user_table, item_table, edge_index):
    all_emb = jnp.concatenate([user_table, item_table], axis=0)
    x0 = jnp.pad(all_emb, ((0, _NPAD - _N), (0, 0)))
    x0 = x0.reshape(_NPAD, 2, _HALF).transpose(1, 0, 2).reshape(2 * _NPAD, _HALF)
    nblk_tot = _NCHROWS // _BLK
    src = edge_index[0].reshape(nblk_tot, _BLK, _CHUNK)
    src3 = jnp.concatenate([src, src + _NPAD], axis=0)
    dst2 = edge_index[1].reshape(nblk_tot, _BLK, _CHUNK)
    out_sum, _, _ = _lgcn(x0, src3, dst2)
    final = out_sum.reshape(2, _NPAD, _HALF).transpose(1, 0, 2)
    final = final.reshape(_NPAD, _D)[:_N]
    return final[:_NUM_USERS], final[_NUM_USERS:]


# speedup vs baseline: 21.3918x; 1.7055x over previous
import functools

import jax
import jax.numpy as jnp
from jax import lax
from jax.experimental import pallas as pl
from jax.experimental.pallas import tpu as pltpu
from jax.experimental.pallas import tpu_sc as plsc

_NUM_USERS = 25000
_NUM_ITEMS = 25000
_D = 64
_HALF = 32
_N = _NUM_USERS + _NUM_ITEMS
_E = 800000
_NS = 16
_NPAD = 51200
_RPT = _NPAD // _NS
_WCH = 64
_NWCH = _RPT // _WCH
_NBUF = 4
_CHUNK = 80
_EPT = _E // _NS
_BLK = 25
_NBLK = _EPT // (_CHUNK * _BLK)
_NCHROWS = _E // _CHUNK


def _lgcn_body(x0, src3, dst2, out_sum, za, zb,
               acc, dacc, ones1, srcb, dstb, rows, wbuf, sbuf,
               dinv, gsems, ssems):
    c = lax.axis_index("c")
    s = lax.axis_index("s")
    row0 = s * _RPT
    nbase = c * _NPAD + row0
    blk0_d = s * _NBLK
    blk0_s = c * (_NS * _NBLK) + blk0_d

    f1 = jnp.full((16,), 1.0, jnp.float32)
    f0 = jnp.zeros((16,), jnp.float32)

    def _fill(ref, nrows, vec):
        def f(r, _):
            ref[r, 0:16] = vec
            ref[r, 16:32] = vec
            return 0
        lax.fori_loop(0, nrows, f, 0)

    def _clear_acc_slice():
        _fill(wbuf, _WCH, f0)
        def f(w, _):
            pltpu.sync_copy(wbuf, acc.at[pl.ds(row0 + w * _WCH, _WCH)])
            return 0
        lax.fori_loop(0, _NWCH, f, 0)

    def _edge_pass(zsrc):
        def blk(b, _):
            pltpu.sync_copy(dst2.at[blk0_d + b], dstb)
            if zsrc is None:
                descs = [
                    pltpu.async_copy(ones1, dacc.at[dstb.at[j]],
                                     ssems.at[j % _NBUF], add=True)
                    for j in range(_BLK)
                ]
                for d in descs:
                    d.wait()
            else:
                pltpu.sync_copy(src3.at[blk0_s + b], srcb)
                def gather(j):
                    return pltpu.async_copy(
                        zsrc.at[srcb.at[j]], rows.at[j % _NBUF],
                        gsems.at[j % _NBUF])
                def scatter(j):
                    return pltpu.async_copy(
                        rows.at[j % _NBUF], acc.at[dstb.at[j]],
                        ssems.at[j % _NBUF], add=True)
                gd = {j: gather(j) for j in range(_NBUF - 1)}
                sd = {}
                for j in range(_BLK):
                    if j + _NBUF - 1 < _BLK:
                        if j >= 1:
                            sd[j - 1].wait()
                        gd[j + _NBUF - 1] = gather(j + _NBUF - 1)
                    gd[j].wait()
                    sd[j] = scatter(j)
                for j in range(max(_BLK - _NBUF, 0), _BLK):
                    sd[j].wait()
            return 0
        lax.fori_loop(0, _NBLK, blk, 0)

    def _dinv_phase():
        magic = jnp.full((16,), 0x5F3759DF, jnp.int32)
        one_i = jnp.full((16,), 1, jnp.int32)
        pltpu.sync_copy(dacc.at[pl.ds(row0, _RPT)], dinv)
        def gf(g, _):
            d = dinv[pl.ds(g * 16, 16)]
            ib = lax.bitcast_convert_type(d, jnp.int32)
            y = lax.bitcast_convert_type(
                magic - lax.shift_right_logical(ib, one_i), jnp.float32)
            for _i in range(4):
                y = y * (1.5 - 0.5 * d * y * y)
            y = jnp.where(d > 0.5, y, 0.0)
            dinv[pl.ds(g * 16, 16)] = y
            return 0
        lax.fori_loop(0, _RPT // 16, gf, 0)

    def _z0_phase():
        def wchunk(w, _):
            nb = nbase + w * _WCH
            pltpu.sync_copy(x0.at[pl.ds(nb, _WCH)], wbuf)
            pltpu.sync_copy(wbuf, out_sum.at[pl.ds(nb, _WCH)])
            def gf(g, _):
                dvec = dinv[pl.ds(w * _WCH + g * 16, 16)]
                for r16 in range(16):
                    r = g * 16 + r16
                    di = dvec[r16]
                    wbuf[r, 0:16] = wbuf[r, 0:16] * di
                    wbuf[r, 16:32] = wbuf[r, 16:32] * di
                return 0
            lax.fori_loop(0, _WCH // 16, gf, 0)
            pltpu.sync_copy(wbuf, za.at[pl.ds(nb, _WCH)])
            return 0
        lax.fori_loop(0, _NWCH, wchunk, 0)

    def _writeback(last, zdst):
        def wchunk(w, _):
            nb = nbase + w * _WCH
            pltpu.sync_copy(acc.at[pl.ds(row0 + w * _WCH, _WCH)], wbuf)
            pltpu.sync_copy(out_sum.at[pl.ds(nb, _WCH)], sbuf)
            def gf(g, _):
                dvec = dinv[pl.ds(w * _WCH + g * 16, 16)]
                for r16 in range(16):
                    r = g * 16 + r16
                    di = dvec[r16]
                    s0 = sbuf[r, 0:16] + wbuf[r, 0:16] * di
                    s1 = sbuf[r, 16:32] + wbuf[r, 16:32] * di
                    if last:
                        sbuf[r, 0:16] = s0 * 0.25
                        sbuf[r, 16:32] = s1 * 0.25
                    else:
                        sbuf[r, 0:16] = s0
                        sbuf[r, 16:32] = s1
                        d2 = di * di
                        wbuf[r, 0:16] = wbuf[r, 0:16] * d2
                        wbuf[r, 16:32] = wbuf[r, 16:32] * d2
                return 0
            lax.fori_loop(0, _WCH // 16, gf, 0)
            pltpu.sync_copy(sbuf, out_sum.at[pl.ds(nb, _WCH)])
            if not last:
                pltpu.sync_copy(wbuf, zdst.at[pl.ds(nb, _WCH)])
            return 0
        lax.fori_loop(0, _NWCH, wchunk, 0)

    for k in range(_CHUNK // 16):
        ones1[pl.ds(k * 16, 16)] = f1
    def zf(g, _):
        dinv[pl.ds(g * 16, 16)] = f0
        return 0
    lax.fori_loop(0, _RPT // 16, zf, 0)
    pltpu.sync_copy(dinv, dacc.at[pl.ds(row0, _RPT)])
    plsc.subcore_barrier()
    _edge_pass(None)
    plsc.subcore_barrier()
    _dinv_phase()
    _z0_phase()

    zsrc = za
    for l in range(3):
        _clear_acc_slice()
        plsc.subcore_barrier()
        _edge_pass(zsrc)
        plsc.subcore_barrier()
        zdst = zb if zsrc is za else za
        _writeback(last=(l == 2), zdst=zdst)
        zsrc = zdst


_lgcn = functools.partial(
    pl.kernel,
    out_type=(
        jax.ShapeDtypeStruct((2 * _NPAD, _HALF), jnp.float32),
        jax.ShapeDtypeStruct((2 * _NPAD, _HALF), jnp.float32),
        jax.ShapeDtypeStruct((2 * _NPAD, _HALF), jnp.float32),
    ),
    mesh=plsc.VectorSubcoreMesh(core_axis_name="c", subcore_axis_name="s"),
    compiler_params=pltpu.CompilerParams(use_tc_tiling_on_sc=False),
    scratch_types=[
        pltpu.VMEM_SHARED((_NPAD, _HALF), jnp.float32),
        pltpu.VMEM_SHARED((_NPAD,), jnp.float32),
        pltpu.VMEM((_CHUNK,), jnp.float32),
        pltpu.VMEM((_BLK, _CHUNK), jnp.int32),
        pltpu.VMEM((_BLK, _CHUNK), jnp.int32),
        pltpu.VMEM((_NBUF, _CHUNK, _HALF), jnp.float32),
        pltpu.VMEM((_WCH, _HALF), jnp.float32),
        pltpu.VMEM((_WCH, _HALF), jnp.float32),
        pltpu.VMEM((_RPT,), jnp.float32),
        pltpu.SemaphoreType.DMA((_NBUF,)),
        pltpu.SemaphoreType.DMA((_NBUF,)),
    ],
)(_lgcn_body)


def kernel(user_table, item_table, edge_index):
    all_emb = jnp.concatenate([user_table, item_table], axis=0)
    x0 = jnp.pad(all_emb, ((0, _NPAD - _N), (0, 0)))
    x0 = x0.reshape(_NPAD, 2, _HALF).transpose(1, 0, 2).reshape(2 * _NPAD, _HALF)
    nblk_tot = _NCHROWS // _BLK
    src = edge_index[0].reshape(nblk_tot, _BLK, _CHUNK)
    src3 = jnp.concatenate([src, src + _NPAD], axis=0)
    dst2 = edge_index[1].reshape(nblk_tot, _BLK, _CHUNK)
    out_sum, _, _ = _lgcn(x0, src3, dst2)
    final = out_sum.reshape(2, _NPAD, _HALF).transpose(1, 0, 2)
    final = final.reshape(_NPAD, _D)[:_N]
    return final[:_NUM_USERS], final[_NUM_USERS:]


# speedup vs baseline: 26.6337x; 1.2450x over previous
import functools

import jax
import jax.numpy as jnp
from jax import lax
from jax.experimental import pallas as pl
from jax.experimental.pallas import tpu as pltpu
from jax.experimental.pallas import tpu_sc as plsc

_NUM_USERS = 25000
_NUM_ITEMS = 25000
_D = 64
_HALF = 32
_N = _NUM_USERS + _NUM_ITEMS
_E = 800000
_NS = 16
_NPAD = 51200
_RPT = _NPAD // _NS
_WCH = 80
_NWCH = _RPT // _WCH
_NBUF = 5
_CHUNK = 80
_EPT = _E // _NS
_BLK = 25
_NBLK = _EPT // (_CHUNK * _BLK)
_NCHROWS = _E // _CHUNK


def _lgcn_body(x0, src3, dst2, out_sum, za, zb,
               acc, dacc, ones1, srcb, dstb, rows, dinv,
               gsems, ssems, isems):
    c = lax.axis_index("c")
    s = lax.axis_index("s")
    row0 = s * _RPT
    nbase = c * _NPAD + row0
    blk0_d = s * _NBLK
    blk0_s = c * (_NS * _NBLK) + blk0_d

    f1 = jnp.full((16,), 1.0, jnp.float32)
    f0 = jnp.zeros((16,), jnp.float32)
    wb0 = rows.at[0]
    wb1 = rows.at[1]

    def _clear_acc_slice():
        def zf(r, _):
            wb0[r, 0:16] = f0
            wb0[r, 16:32] = f0
            return 0
        lax.fori_loop(0, _WCH, zf, 0)
        def f(w, _):
            pltpu.sync_copy(wb0, acc.at[pl.ds(row0 + w * _WCH, _WCH)])
            return 0
        lax.fori_loop(0, _NWCH, f, 0)

    def _edge_pass(zsrc):
        deg = zsrc is None

        def load_idx(setk, b):
            ds_ = [pltpu.async_copy(dst2.at[blk0_d + b], dstb.at[setk],
                                    isems.at[0])]
            if not deg:
                ds_.append(pltpu.async_copy(src3.at[blk0_s + b], srcb.at[setk],
                                            isems.at[1]))
            return ds_

        def process(setk, b):
            sb = srcb.at[setk]
            db = dstb.at[setk]
            if deg:
                descs = [pltpu.async_copy(ones1, dacc.at[db.at[j]],
                                          ssems.at[j % _NBUF], add=True)
                         for j in range(_BLK)]
                for d in descs:
                    d.wait()
            else:
                def gather(j):
                    return pltpu.async_copy(zsrc.at[sb.at[j]],
                                            rows.at[j % _NBUF],
                                            gsems.at[j % _NBUF])
                def scatter(j):
                    return pltpu.async_copy(rows.at[j % _NBUF],
                                            acc.at[db.at[j]],
                                            ssems.at[j % _NBUF], add=True)
                gd = {j: gather(j) for j in range(_NBUF - 1)}
                sd = {}
                for j in range(_BLK):
                    if j + _NBUF - 1 < _BLK:
                        if j >= 1:
                            sd[j - 1].wait()
                        gd[j + _NBUF - 1] = gather(j + _NBUF - 1)
                    gd[j].wait()
                    sd[j] = scatter(j)
                for j in range(max(_BLK - _NBUF, 0), _BLK):
                    sd[j].wait()

        for d in load_idx(0, 0):
            d.wait()
        def pair(p, _):
            b0 = 2 * p
            d1 = load_idx(1, b0 + 1)
            process(0, b0)
            for d in d1:
                d.wait()
            d0 = load_idx(0, b0 + 2)
            process(1, b0 + 1)
            for d in d0:
                d.wait()
            return 0
        lax.fori_loop(0, _NBLK // 2, pair, 0)
        process(0, _NBLK - 1)

    def _dinv_phase():
        magic = jnp.full((16,), 0x5F3759DF, jnp.int32)
        one_i = jnp.full((16,), 1, jnp.int32)
        pltpu.sync_copy(dacc.at[pl.ds(row0, _RPT)], dinv)
        def gf(g, _):
            d = dinv[pl.ds(g * 16, 16)]
            ib = lax.bitcast_convert_type(d, jnp.int32)
            y = lax.bitcast_convert_type(
                magic - lax.shift_right_logical(ib, one_i), jnp.float32)
            for _i in range(4):
                y = y * (1.5 - 0.5 * d * y * y)
            y = jnp.where(d > 0.5, y, 0.0)
            dinv[pl.ds(g * 16, 16)] = y
            return 0
        lax.fori_loop(0, _RPT // 16, gf, 0)

    def _z0_phase():
        def wchunk(w, _):
            nb = nbase + w * _WCH
            pltpu.sync_copy(x0.at[pl.ds(nb, _WCH)], wb0)
            pltpu.sync_copy(wb0, out_sum.at[pl.ds(nb, _WCH)])
            def gf(g, _):
                dvec = dinv[pl.ds(w * _WCH + g * 16, 16)]
                for r16 in range(16):
                    r = g * 16 + r16
                    di = dvec[r16]
                    wb0[r, 0:16] = wb0[r, 0:16] * di
                    wb0[r, 16:32] = wb0[r, 16:32] * di
                return 0
            lax.fori_loop(0, _WCH // 16, gf, 0)
            pltpu.sync_copy(wb0, za.at[pl.ds(nb, _WCH)])
            return 0
        lax.fori_loop(0, _NWCH, wchunk, 0)

    def _writeback(last, zdst):
        def wchunk(w, _):
            nb = nbase + w * _WCH
            da = pltpu.async_copy(acc.at[pl.ds(row0 + w * _WCH, _WCH)], wb0,
                                  gsems.at[0])
            db = pltpu.async_copy(out_sum.at[pl.ds(nb, _WCH)], wb1,
                                  gsems.at[1])
            da.wait()
            db.wait()
            def gf(g, _):
                dvec = dinv[pl.ds(w * _WCH + g * 16, 16)]
                for r16 in range(16):
                    r = g * 16 + r16
                    di = dvec[r16]
                    s0 = wb1[r, 0:16] + wb0[r, 0:16] * di
                    s1 = wb1[r, 16:32] + wb0[r, 16:32] * di
                    if last:
                        wb1[r, 0:16] = s0 * 0.25
                        wb1[r, 16:32] = s1 * 0.25
                    else:
                        wb1[r, 0:16] = s0
                        wb1[r, 16:32] = s1
                        d2 = di * di
                        wb0[r, 0:16] = wb0[r, 0:16] * d2
                        wb0[r, 16:32] = wb0[r, 16:32] * d2
                return 0
            lax.fori_loop(0, _WCH // 16, gf, 0)
            pltpu.sync_copy(wb1, out_sum.at[pl.ds(nb, _WCH)])
            if not last:
                pltpu.sync_copy(wb0, zdst.at[pl.ds(nb, _WCH)])
            return 0
        lax.fori_loop(0, _NWCH, wchunk, 0)

    for k in range(_CHUNK // 16):
        ones1[pl.ds(k * 16, 16)] = f1
    def zf(g, _):
        dinv[pl.ds(g * 16, 16)] = f0
        return 0
    lax.fori_loop(0, _RPT // 16, zf, 0)
    pltpu.sync_copy(dinv, dacc.at[pl.ds(row0, _RPT)])
    plsc.subcore_barrier()
    _edge_pass(None)
    plsc.subcore_barrier()
    _dinv_phase()
    _z0_phase()

    zsrc = za
    for l in range(3):
        _clear_acc_slice()
        plsc.subcore_barrier()
        _edge_pass(zsrc)
        plsc.subcore_barrier()
        zdst = zb if zsrc is za else za
        _writeback(last=(l == 2), zdst=zdst)
        zsrc = zdst


_lgcn = functools.partial(
    pl.kernel,
    out_type=(
        jax.ShapeDtypeStruct((2 * _NPAD, _HALF), jnp.float32),
        jax.ShapeDtypeStruct((2 * _NPAD, _HALF), jnp.float32),
        jax.ShapeDtypeStruct((2 * _NPAD, _HALF), jnp.float32),
    ),
    mesh=plsc.VectorSubcoreMesh(core_axis_name="c", subcore_axis_name="s"),
    compiler_params=pltpu.CompilerParams(use_tc_tiling_on_sc=False),
    scratch_types=[
        pltpu.VMEM_SHARED((_NPAD, _HALF), jnp.float32),
        pltpu.VMEM_SHARED((_NPAD,), jnp.float32),
        pltpu.VMEM((_CHUNK,), jnp.float32),
        pltpu.VMEM((2, _BLK, _CHUNK), jnp.int32),
        pltpu.VMEM((2, _BLK, _CHUNK), jnp.int32),
        pltpu.VMEM((_NBUF, _CHUNK, _HALF), jnp.float32),
        pltpu.VMEM((_RPT,), jnp.float32),
        pltpu.SemaphoreType.DMA((_NBUF,)),
        pltpu.SemaphoreType.DMA((_NBUF,)),
        pltpu.SemaphoreType.DMA((2,)),
    ],
)(_lgcn_body)


def kernel(user_table, item_table, edge_index):
    all_emb = jnp.concatenate([user_table, item_table], axis=0)
    x0 = jnp.pad(all_emb, ((0, _NPAD - _N), (0, 0)))
    x0 = x0.reshape(_NPAD, 2, _HALF).transpose(1, 0, 2).reshape(2 * _NPAD, _HALF)
    nblk_tot = _NCHROWS // _BLK
    src = edge_index[0].reshape(nblk_tot, _BLK, _CHUNK)
    src3 = jnp.concatenate([src, src + _NPAD], axis=0)
    dst2 = edge_index[1].reshape(nblk_tot, _BLK, _CHUNK)
    out_sum, _, _ = _lgcn(x0, src3, dst2)
    final = out_sum.reshape(2, _NPAD, _HALF).transpose(1, 0, 2)
    final = final.reshape(_NPAD, _D)[:_N]
    return final[:_NUM_USERS], final[_NUM_USERS:]
